# baseline (speedup 1.0000x reference)
import numpy as np

import jax
import jax.numpy as jnp
from jax.experimental import pallas as pl
from jax.experimental.pallas import tpu as pltpu


def _expanded_selectors(n_in, n_pool, n_ch, k):
    N = n_pool * n_pool * n_ch
    n = np.arange(N)
    b, d, m = n // (n_pool * n_ch), (n // n_ch) % n_pool, n % n_ch
    ac = np.arange(2)[:, None, None, None]
    kk = np.arange(k)[None, :, None, None]
    h = np.arange(n_in)[None, None, :, None]
    u = (h == 2 * b[None, None, None, :] + ac + kk).astype(np.float32)
    v = (h == 2 * d[None, None, None, :] + ac + kk).astype(np.float32)
    oh = (np.arange(n_ch)[:, None] == m[None, :]).astype(np.float32)
    return u, v, oh


_U1, _V1, _OH1 = _expanded_selectors(28, 12, 6, 5)
_U2, _V2, _OH2 = _expanded_selectors(12, 4, 16, 5)


def _conv1_quarters(w1p):
    w1e = jnp.dot(w1p[:25, :6], _OH1).reshape(5, 5, 864)
    out = []
    for a in range(2):
        for c in range(2):
            t = [sum(_V1[c, l] * w1e[k, l][None, :] for l in range(5))
                 for k in range(5)]
            w = sum(_U1[a, k][:, None, :] * t[k][None, :, :] for k in range(5))
            out.append(w.reshape(784, 864).astype(jnp.bfloat16))
    return out


def _conv2_quarters(w2p):
    w2e = jnp.dot(w2p[:150, :16].reshape(150, 16), _OH2)
    w2e = w2e.reshape(6, 5, 5, 256)
    out = []
    for a in range(2):
        for c in range(2):
            t = [sum(_V2[c, l][:, None, :] * w2e[:, k, l][None, :, :]
                     for l in range(5))
                 for k in range(5)]
            w = sum(_U2[a, k][:, None, None, :] * t[k][None, :, :, :]
                    for k in range(5))
            out.append(w.reshape(864, 256).astype(jnp.bfloat16))
    return out


def _lenet_kernel(x_ref, w1q0, w1q1, w1q2, w1q3, b1_ref,
                  w2q0, w2q1, w2q2, w2q3, b2_ref,
                  f1_ref, fb1_ref, f2_ref, fb2_ref, o_ref):
    tile = x_ref.shape[0]
    half = tile // 2
    b1 = b1_ref[...]
    b2 = b2_ref[...]
    w1 = (w1q0[...], w1q1[...], w1q2[...], w1q3[...])
    w2 = (w2q0[...], w2q1[...], w2q2[...], w2q3[...])
    f1 = f1_ref[...]
    f2 = f2_ref[...]
    fb1v = fb1_ref[...]
    fb2v = fb2_ref[...]

    xs = [x_ref[i * half:(i + 1) * half]
          .reshape(half, 784).astype(jnp.bfloat16) for i in range(2)]
    for i in range(2):
        x = xs[i]
        p1 = jnp.tanh(jnp.dot(x, w1[0], preferred_element_type=jnp.float32) + b1)
        p1 += jnp.tanh(jnp.dot(x, w1[1], preferred_element_type=jnp.float32) + b1)
        p1 += jnp.tanh(jnp.dot(x, w1[2], preferred_element_type=jnp.float32) + b1)
        p1 += jnp.tanh(jnp.dot(x, w1[3], preferred_element_type=jnp.float32) + b1)
        p1 = (0.25 * p1).astype(jnp.bfloat16)

        p2 = jnp.tanh(jnp.dot(p1, w2[0], preferred_element_type=jnp.float32) + b2)
        p2 += jnp.tanh(jnp.dot(p1, w2[1], preferred_element_type=jnp.float32) + b2)
        p2 += jnp.tanh(jnp.dot(p1, w2[2], preferred_element_type=jnp.float32) + b2)
        p2 += jnp.tanh(jnp.dot(p1, w2[3], preferred_element_type=jnp.float32) + b2)
        p2 = (0.25 * p2).astype(jnp.bfloat16)

        h = jnp.tanh(
            jnp.dot(p2, f1, preferred_element_type=jnp.float32) + fb1v
        ).astype(jnp.bfloat16)
        o_ref[i * half:(i + 1) * half, :] = (
            jnp.dot(h, f2, preferred_element_type=jnp.float32) + fb2v
        )


def kernel(x, w1p, b1p, w2p, b2p, f1p, fb1, f2p, fb2):
    B = x.shape[0]

    w1q = _conv1_quarters(w1p)
    w2q = _conv2_quarters(w2p)
    bias1 = jnp.tile(b1p[:1, :6], (1, 144))
    bias2 = jnp.tile(b2p[:1, :16], (1, 16))
    f1b = f1p.astype(jnp.bfloat16)
    f2b = f2p.astype(jnp.bfloat16)

    tile = 1024 if B % 1024 == 0 else B
    const = lambda i: (0, 0)
    out = pl.pallas_call(
        _lenet_kernel,
        out_shape=jax.ShapeDtypeStruct((B, 128), jnp.float32),
        grid_spec=pltpu.PrefetchScalarGridSpec(
            num_scalar_prefetch=0,
            grid=(B // tile,),
            in_specs=[
                pl.BlockSpec((tile, 28, 28), lambda i: (i, 0, 0)),
                pl.BlockSpec((784, 864), const),
                pl.BlockSpec((784, 864), const),
                pl.BlockSpec((784, 864), const),
                pl.BlockSpec((784, 864), const),
                pl.BlockSpec((1, 864), const),
                pl.BlockSpec((864, 256), const),
                pl.BlockSpec((864, 256), const),
                pl.BlockSpec((864, 256), const),
                pl.BlockSpec((864, 256), const),
                pl.BlockSpec((1, 256), const),
                pl.BlockSpec((256, 128), const),
                pl.BlockSpec((1, 128), const),
                pl.BlockSpec((128, 128), const),
                pl.BlockSpec((1, 128), const),
            ],
            out_specs=pl.BlockSpec((tile, 128), lambda i: (i, 0)),
        ),
        compiler_params=pltpu.CompilerParams(
            dimension_semantics=("arbitrary",)),
    )(x.reshape(B, 28, 28), *w1q, bias1, *w2q, bias2, f1b, fb1, f2b, fb2)
    return out[:, :84]


# speedup vs baseline: 1.0240x; 1.0240x over previous
import numpy as np

import jax
import jax.numpy as jnp
from jax.experimental import pallas as pl
from jax.experimental.pallas import tpu as pltpu


def _expanded_selectors(n_in, n_pool, n_ch, k):
    N = n_pool * n_pool * n_ch
    n = np.arange(N)
    b, d, m = n // (n_pool * n_ch), (n // n_ch) % n_pool, n % n_ch
    ac = np.arange(2)[:, None, None, None]
    kk = np.arange(k)[None, :, None, None]
    h = np.arange(n_in)[None, None, :, None]
    u = (h == 2 * b[None, None, None, :] + ac + kk).astype(np.float32)
    v = (h == 2 * d[None, None, None, :] + ac + kk).astype(np.float32)
    oh = (np.arange(n_ch)[:, None] == m[None, :]).astype(np.float32)
    return u, v, oh


_U1, _V1, _OH1 = _expanded_selectors(28, 12, 6, 5)
_U2, _V2, _OH2 = _expanded_selectors(12, 4, 16, 5)


def _conv1_quarters(w1p):
    w1e = jnp.dot(w1p[:25, :6], _OH1).reshape(5, 5, 864)
    out = []
    for a in range(2):
        for c in range(2):
            t = [sum(_V1[c, l] * w1e[k, l][None, :] for l in range(5))
                 for k in range(5)]
            w = sum(_U1[a, k][:, None, :] * t[k][None, :, :] for k in range(5))
            out.append(w.reshape(784, 864).astype(jnp.bfloat16))
    return out


def _conv2_quarters(w2p):
    w2e = jnp.dot(w2p[:150, :16].reshape(150, 16), _OH2)
    w2e = w2e.reshape(6, 5, 5, 256)
    out = []
    for a in range(2):
        for c in range(2):
            t = [sum(_V2[c, l][:, None, :] * w2e[:, k, l][None, :, :]
                     for l in range(5))
                 for k in range(5)]
            w = sum(_U2[a, k][:, None, None, :] * t[k][None, :, :, :]
                    for k in range(5))
            out.append(w.reshape(864, 256).astype(jnp.bfloat16))
    return out


def _lenet_kernel(x_ref, w1q0, w1q1, w1q2, w1q3, b1_ref,
                  w2q0, w2q1, w2q2, w2q3, b2_ref,
                  f1_ref, fb1_ref, f2_ref, fb2_ref, o_ref):
    tile = x_ref.shape[0]
    x = x_ref[...].reshape(tile, 784).astype(jnp.bfloat16)

    b1 = b1_ref[...]
    p1 = jnp.tanh(jnp.dot(x, w1q0[...], preferred_element_type=jnp.float32) + b1)
    p1 += jnp.tanh(jnp.dot(x, w1q1[...], preferred_element_type=jnp.float32) + b1)
    p1 += jnp.tanh(jnp.dot(x, w1q2[...], preferred_element_type=jnp.float32) + b1)
    p1 += jnp.tanh(jnp.dot(x, w1q3[...], preferred_element_type=jnp.float32) + b1)
    p1 = (0.25 * p1).astype(jnp.bfloat16)

    b2 = b2_ref[...]
    p2 = jnp.tanh(jnp.dot(p1, w2q0[...], preferred_element_type=jnp.float32) + b2)
    p2 += jnp.tanh(jnp.dot(p1, w2q1[...], preferred_element_type=jnp.float32) + b2)
    p2 += jnp.tanh(jnp.dot(p1, w2q2[...], preferred_element_type=jnp.float32) + b2)
    p2 += jnp.tanh(jnp.dot(p1, w2q3[...], preferred_element_type=jnp.float32) + b2)
    p2 = (0.25 * p2).astype(jnp.bfloat16)

    h = jnp.tanh(
        jnp.dot(p2, f1_ref[...], preferred_element_type=jnp.float32)
        + fb1_ref[...]
    ).astype(jnp.bfloat16)
    o_ref[...] = (
        jnp.dot(h, f2_ref[...], preferred_element_type=jnp.float32)
        + fb2_ref[...]
    )


def kernel(x, w1p, b1p, w2p, b2p, f1p, fb1, f2p, fb2):
    B = x.shape[0]

    w1q = _conv1_quarters(w1p)
    w2q = _conv2_quarters(w2p)
    bias1 = jnp.tile(b1p[:1, :6], (1, 144))
    bias2 = jnp.tile(b2p[:1, :16], (1, 16))
    f1b = f1p.astype(jnp.bfloat16)
    f2b = f2p.astype(jnp.bfloat16)

    tile = 1024 if B % 1024 == 0 else B
    const = lambda i: (0, 0)
    out = pl.pallas_call(
        _lenet_kernel,
        out_shape=jax.ShapeDtypeStruct((B, 128), jnp.float32),
        grid_spec=pltpu.PrefetchScalarGridSpec(
            num_scalar_prefetch=0,
            grid=(B // tile,),
            in_specs=[
                pl.BlockSpec((tile, 28, 28), lambda i: (i, 0, 0)),
                pl.BlockSpec((784, 864), const),
                pl.BlockSpec((784, 864), const),
                pl.BlockSpec((784, 864), const),
                pl.BlockSpec((784, 864), const),
                pl.BlockSpec((1, 864), const),
                pl.BlockSpec((864, 256), const),
                pl.BlockSpec((864, 256), const),
                pl.BlockSpec((864, 256), const),
                pl.BlockSpec((864, 256), const),
                pl.BlockSpec((1, 256), const),
                pl.BlockSpec((256, 128), const),
                pl.BlockSpec((1, 128), const),
                pl.BlockSpec((128, 128), const),
                pl.BlockSpec((1, 128), const),
            ],
            out_specs=pl.BlockSpec((tile, 128), lambda i: (i, 0)),
        ),
        compiler_params=pltpu.CompilerParams(
            dimension_semantics=("arbitrary",)),
    )(x.reshape(B, 28, 28), *w1q, bias1, *w2q, bias2, f1b, fb1, f2b, fb2)
    return out[:, :84]
